# baseline (speedup 1.0000x reference)
import functools

import jax
import jax.numpy as jnp
from jax import lax
from jax.experimental import pallas as pl
from jax.experimental.pallas import tpu as pltpu, tpu_sc as plsc

SIZE_VOCAB = 1000000
D = 64
B = 16384
L = 50

NC = 2
NS = 16
LANES = 16
NW = NC * NS
BC = B // NW
NCH = BC // 128


def _rsqrt(x):
    i = lax.bitcast_convert_type(x, jnp.int32)
    y = lax.bitcast_convert_type(
        jnp.int32(0x5F3759DF) - lax.shift_right_arithmetic(i, 1), jnp.float32)
    for _ in range(3):
        y = y * (1.5 - 0.5 * x * y * y)
    return y


def _body(cen_hbm, ctx_hbm, win_hbm, wout_hbm, out_hbm,
          cidx_v, ctxidx_v, in_v, out_v, invin_v, res_v, sem):
    wid = lax.axis_index("s") * NC + lax.axis_index("c")
    base = wid * BC

    lanes = lax.iota(jnp.int32, LANES)

    pltpu.sync_copy(cen_hbm.at[wid], cidx_v)
    pltpu.sync_copy(ctx_hbm.at[wid], ctxidx_v)

    for j in range(NCH):
        pltpu.async_copy(win_hbm.at[cidx_v.at[j]],
                         in_v.at[pl.ds(j * 128, 128), :], sem)
    for j in range(NCH):
        pltpu.make_async_copy(win_hbm.at[cidx_v.at[j]],
                              in_v.at[pl.ds(j * 128, 128), :], sem).wait()

    def norm_body(g, _):
        rows = g * LANES + lanes
        acc = jnp.zeros((LANES,), jnp.float32)
        for d in range(D):
            v = plsc.load_gather(in_v, [rows, jnp.full((LANES,), d, jnp.int32)])
            acc += v * v
        invin_v[pl.ds(g * LANES, LANES)] = _rsqrt(acc)
        return ()

    lax.fori_loop(0, BC // LANES, norm_body, (), unroll=False)

    def l_body(l, _):
        for j in range(NCH):
            pltpu.async_copy(wout_hbm.at[ctxidx_v.at[l, j]],
                             out_v.at[pl.ds(j * 128, 128), :], sem)
        for j in range(NCH):
            pltpu.make_async_copy(wout_hbm.at[ctxidx_v.at[l, j]],
                                  out_v.at[pl.ds(j * 128, 128), :],
                                  sem).wait()

        def g_body(g, _):
            rows = g * LANES + lanes
            acc_d = jnp.zeros((LANES,), jnp.float32)
            acc_s = jnp.zeros((LANES,), jnp.float32)
            for d in range(D):
                col = jnp.full((LANES,), d, jnp.int32)
                o = plsc.load_gather(out_v, [rows, col])
                i = plsc.load_gather(in_v, [rows, col])
                acc_d += o * i
                acc_s += o * o
            res = acc_d * _rsqrt(acc_s) * invin_v[pl.ds(g * LANES, LANES)]
            res_v[pl.ds(g * LANES, LANES)] = res
            return ()

        lax.fori_loop(0, BC // LANES, g_body, (), unroll=False)
        pltpu.sync_copy(res_v, out_hbm.at[l, pl.ds(base, BC)])
        return ()

    lax.fori_loop(0, L, l_body, (), unroll=False)


@jax.jit
def kernel(center, context, emb_in_weight, emb_out_weight):
    cen = center.reshape(NW, NCH, 128).astype(jnp.int32)
    ctx = (context.reshape(L, NW, BC).transpose(1, 0, 2)
           .reshape(NW, L, NCH, 128).astype(jnp.int32))

    mesh = plsc.VectorSubcoreMesh(core_axis_name="c", subcore_axis_name="s")
    f = pl.kernel(
        _body,
        out_type=jax.ShapeDtypeStruct((L, B), jnp.float32),
        mesh=mesh,
        compiler_params=pltpu.CompilerParams(needs_layout_passes=False, use_tc_tiling_on_sc=False),
        scratch_types=[
            pltpu.VMEM((NCH, 128), jnp.int32),
            pltpu.VMEM((L, NCH, 128), jnp.int32),
            pltpu.VMEM((BC, D), jnp.float32),
            pltpu.VMEM((BC, D), jnp.float32),
            pltpu.VMEM((BC,), jnp.float32),
            pltpu.VMEM((BC,), jnp.float32),
            pltpu.SemaphoreType.DMA,
        ],
    )
    return f(cen, ctx, emb_in_weight, emb_out_weight)


# speedup vs baseline: 1.7490x; 1.7490x over previous
import functools

import jax
import jax.numpy as jnp
from jax import lax
from jax.experimental import pallas as pl
from jax.experimental.pallas import tpu as pltpu, tpu_sc as plsc

SIZE_VOCAB = 1000000
D = 64
B = 16384
L = 50

NC = 2
NS = 16
LANES = 16
NW = NC * NS
BC = B // NW
NCH = BC // 128


def _rsqrt(x):
    i = lax.bitcast_convert_type(x, jnp.int32)
    y = lax.bitcast_convert_type(
        jnp.int32(0x5F3759DF) - lax.shift_right_arithmetic(i, 1), jnp.float32)
    for _ in range(3):
        y = y * (1.5 - 0.5 * x * y * y)
    return y


def _body(cen_hbm, ctx_hbm, win_hbm, wout_hbm, out_hbm,
          cidx_v, ctxidx_v, in_v, out_v, invin_v, res_v, sem):
    wid = lax.axis_index("s") * NC + lax.axis_index("c")
    base = wid * BC

    lanes = lax.iota(jnp.int32, LANES)

    pltpu.sync_copy(cen_hbm.at[wid], cidx_v)
    pltpu.sync_copy(ctx_hbm.at[wid], ctxidx_v)

    for j in range(NCH):
        pltpu.async_copy(win_hbm.at[cidx_v.at[j]],
                         in_v.at[pl.ds(j * 128, 128), :], sem)
    for j in range(NCH):
        pltpu.make_async_copy(win_hbm.at[cidx_v.at[j]],
                              in_v.at[pl.ds(j * 128, 128), :], sem).wait()

    def norm_body(g, _):
        rows = g * LANES + lanes
        acc = jnp.zeros((LANES,), jnp.float32)
        for d in range(D):
            col = (lanes + d) & (D - 1)
            v = plsc.load_gather(in_v, [rows, col])
            acc += v * v
        invin_v[pl.ds(g * LANES, LANES)] = _rsqrt(acc)
        return ()

    lax.fori_loop(0, BC // LANES, norm_body, (), unroll=False)

    def l_body(l, _):
        for j in range(NCH):
            pltpu.async_copy(wout_hbm.at[ctxidx_v.at[l, j]],
                             out_v.at[pl.ds(j * 128, 128), :], sem)
        for j in range(NCH):
            pltpu.make_async_copy(wout_hbm.at[ctxidx_v.at[l, j]],
                                  out_v.at[pl.ds(j * 128, 128), :],
                                  sem).wait()

        def g_body(g, _):
            rows = g * LANES + lanes
            acc_d = jnp.zeros((LANES,), jnp.float32)
            acc_s = jnp.zeros((LANES,), jnp.float32)
            for d in range(D):
                col = (lanes + d) & (D - 1)
                o = plsc.load_gather(out_v, [rows, col])
                i = plsc.load_gather(in_v, [rows, col])
                acc_d += o * i
                acc_s += o * o
            res = acc_d * _rsqrt(acc_s) * invin_v[pl.ds(g * LANES, LANES)]
            res_v[pl.ds(g * LANES, LANES)] = res
            return ()

        lax.fori_loop(0, BC // LANES, g_body, (), unroll=False)
        pltpu.sync_copy(res_v, out_hbm.at[l, pl.ds(base, BC)])
        return ()

    lax.fori_loop(0, L, l_body, (), unroll=False)


@jax.jit
def kernel(center, context, emb_in_weight, emb_out_weight):
    cen = center.reshape(NW, NCH, 128).astype(jnp.int32)
    ctx = (context.reshape(L, NW, BC).transpose(1, 0, 2)
           .reshape(NW, L, NCH, 128).astype(jnp.int32))

    mesh = plsc.VectorSubcoreMesh(core_axis_name="c", subcore_axis_name="s")
    f = pl.kernel(
        _body,
        out_type=jax.ShapeDtypeStruct((L, B), jnp.float32),
        mesh=mesh,
        compiler_params=pltpu.CompilerParams(needs_layout_passes=False, use_tc_tiling_on_sc=False),
        scratch_types=[
            pltpu.VMEM((NCH, 128), jnp.int32),
            pltpu.VMEM((L, NCH, 128), jnp.int32),
            pltpu.VMEM((BC, D), jnp.float32),
            pltpu.VMEM((BC, D), jnp.float32),
            pltpu.VMEM((BC,), jnp.float32),
            pltpu.VMEM((BC,), jnp.float32),
            pltpu.SemaphoreType.DMA,
        ],
    )
    return f(cen, ctx, emb_in_weight, emb_out_weight)
